# baseline (speedup 1.0000x reference)
import math

import jax
import jax.numpy as jnp
from jax import lax
from jax.experimental import pallas as pl
from jax.experimental.pallas import tpu as pltpu
from jax.experimental.pallas import tpu_sc as plsc

D_MODEL = 64
SCALE = math.sqrt(D_MODEL)

NC = 2
NS = 16
NW = NC * NS

C = 128
NBUF = 2


def _scale_chunk(src, dst, idx_v, off):
    @plsc.parallel_loop(0, C // 16, unroll=2)
    def body(g):
        pv = jnp.bitwise_and(idx_v[pl.ds(off + g * 16, 16)], 1) * D_MODEL
        for l in range(16):
            p = pv[l]
            r = g * 16 + l
            for q in range(D_MODEL // 16):
                dst[r, pl.ds(q * 16, 16)] = (
                    src[r, pl.ds(p + q * 16, 16)] * SCALE
                )


def _body(table, idx_hbm, out_hbm, idx_v, jbuf, in_bufs, out_bufs, gsems, osems):
    n_w = idx_hbm.shape[0] // NW
    steps = n_w // C
    wid = lax.axis_index("s") * NC + lax.axis_index("c")
    base = wid * n_w
    pltpu.sync_copy(idx_hbm.at[pl.ds(base, n_w)], idx_v)

    @plsc.parallel_loop(0, n_w // 16, unroll=8)
    def _mkj(i):
        jbuf[pl.ds(i * 16, 16)] = lax.shift_right_logical(
            idx_v[pl.ds(i * 16, 16)], 1
        )

    def fire_gather(s, b):
        pltpu.async_copy(
            table.at[jbuf.at[pl.ds(s * C, C)]], in_bufs.at[b], gsems.at[b]
        )

    def wait_gather(b):
        pltpu.make_async_copy(
            table.at[pl.ds(0, C)], in_bufs.at[b], gsems.at[b]
        ).wait()

    def fire_scatter(s, b):
        pltpu.async_copy(
            out_bufs.at[b], out_hbm.at[pl.ds(base + s * C, C)], osems.at[b]
        )

    def wait_scatter(b):
        pltpu.make_async_copy(
            out_bufs.at[b], out_hbm.at[pl.ds(0, C)], osems.at[b]
        ).wait()

    for b in range(NBUF):
        fire_gather(b, b)

    for b in range(NBUF):
        wait_gather(b)
        _scale_chunk(in_bufs.at[b], out_bufs.at[b], idx_v, b * C)
        fire_gather(b + NBUF, b)
        fire_scatter(b, b)

    def group(g, carry):
        for b in range(NBUF):
            s = g * NBUF + b
            wait_gather(b)
            wait_scatter(b)
            _scale_chunk(in_bufs.at[b], out_bufs.at[b], idx_v, s * C)
            fire_gather(s + NBUF, b)
            fire_scatter(s, b)
        return carry

    lax.fori_loop(1, steps // NBUF - 1, group, 0, unroll=False)

    for b in range(NBUF):
        s = steps - NBUF + b
        wait_gather(b)
        wait_scatter(b)
        _scale_chunk(in_bufs.at[b], out_bufs.at[b], idx_v, s * C)
        fire_scatter(s, b)

    for b in range(NBUF):
        wait_scatter(b)


def _gather_kernel(n_total):
    return pl.kernel(
        _body,
        out_type=jax.ShapeDtypeStruct((n_total, D_MODEL), jnp.float32),
        mesh=plsc.VectorSubcoreMesh(core_axis_name="c", subcore_axis_name="s"),
        scratch_types=[
            pltpu.VMEM((n_total // NW,), jnp.int32),
            pltpu.VMEM((n_total // NW,), jnp.int32),
            pltpu.VMEM((NBUF, C, 128), jnp.float32),
            pltpu.VMEM((NBUF, C, D_MODEL), jnp.float32),
            pltpu.SemaphoreType.DMA((NBUF,)),
            pltpu.SemaphoreType.DMA((NBUF,)),
        ],
        compiler_params=pltpu.CompilerParams(
            use_tc_tiling_on_sc=True, needs_layout_passes=False
        ),
    )


def kernel(x, emb_weight):
    b, s = x.shape
    n_total = b * s
    wpair = emb_weight.reshape(500000, 2 * D_MODEL)
    flat_idx = x.T.reshape(n_total).astype(jnp.int32)
    out = _gather_kernel(n_total)(wpair, flat_idx)
    return jnp.transpose(out.reshape(s, b, D_MODEL), (1, 0, 2))


# speedup vs baseline: 1.1264x; 1.1264x over previous
import math

import jax
import jax.numpy as jnp
from jax import lax
from jax.experimental import pallas as pl
from jax.experimental.pallas import tpu as pltpu
from jax.experimental.pallas import tpu_sc as plsc

D_MODEL = 64
SCALE = math.sqrt(D_MODEL)

NC = 2
NS = 16
NW = NC * NS

C = 128
NBUF = 2


def _scale_chunk(src, dst):
    @plsc.parallel_loop(0, C, unroll=4)
    def body(r):
        for q in range(D_MODEL // 16):
            dst[r, pl.ds(q * 16, 16)] = src[r, pl.ds(q * 16, 16)] * SCALE


def _body(table, idx_hbm, out_hbm, idx_v, in_bufs, out_bufs, gsems, osems):
    n_w = idx_hbm.shape[0] // NW
    steps = n_w // C
    wid = lax.axis_index("s") * NC + lax.axis_index("c")
    base = wid * n_w
    pltpu.sync_copy(idx_hbm.at[pl.ds(base, n_w)], idx_v)

    def fire_gather(s, b):
        pltpu.async_copy(
            table.at[idx_v.at[pl.ds(s * C, C)]], in_bufs.at[b], gsems.at[b]
        )

    def wait_gather(b):
        pltpu.make_async_copy(
            table.at[pl.ds(0, C)], in_bufs.at[b], gsems.at[b]
        ).wait()

    def fire_scatter(s, b):
        pltpu.async_copy(
            out_bufs.at[b], out_hbm.at[pl.ds(base + s * C, C)], osems.at[b]
        )

    def wait_scatter(b):
        pltpu.make_async_copy(
            out_bufs.at[b], out_hbm.at[pl.ds(0, C)], osems.at[b]
        ).wait()

    for b in range(NBUF):
        fire_gather(b, b)

    for b in range(NBUF):
        wait_gather(b)
        _scale_chunk(in_bufs.at[b], out_bufs.at[b])
        fire_gather(b + NBUF, b)
        fire_scatter(b, b)

    def group(g, carry):
        for b in range(NBUF):
            s = g * NBUF + b
            wait_gather(b)
            wait_scatter(b)
            _scale_chunk(in_bufs.at[b], out_bufs.at[b])
            fire_gather(s + NBUF, b)
            fire_scatter(s, b)
        return carry

    lax.fori_loop(1, steps // NBUF - 1, group, 0, unroll=False)

    for b in range(NBUF):
        s = steps - NBUF + b
        wait_gather(b)
        wait_scatter(b)
        _scale_chunk(in_bufs.at[b], out_bufs.at[b])
        fire_scatter(s, b)

    for b in range(NBUF):
        wait_scatter(b)


def _gather_kernel(n_total):
    return pl.kernel(
        _body,
        out_type=jax.ShapeDtypeStruct((n_total, D_MODEL), jnp.float32),
        mesh=plsc.VectorSubcoreMesh(core_axis_name="c", subcore_axis_name="s"),
        scratch_types=[
            pltpu.VMEM((n_total // NW,), jnp.int32),
            pltpu.VMEM((NBUF, C, 128), jnp.float32),
            pltpu.VMEM((NBUF, C, D_MODEL), jnp.float32),
            pltpu.SemaphoreType.DMA((NBUF,)),
            pltpu.SemaphoreType.DMA((NBUF,)),
        ],
        compiler_params=pltpu.CompilerParams(
            use_tc_tiling_on_sc=True, needs_layout_passes=False
        ),
    )


def kernel(x, emb_weight):
    b, s = x.shape
    n_total = b * s
    wpad = jnp.pad(emb_weight, ((0, 0), (0, 128 - D_MODEL)))
    flat_idx = x.T.reshape(n_total).astype(jnp.int32)
    out = _gather_kernel(n_total)(wpad, flat_idx)
    return jnp.transpose(out.reshape(s, b, D_MODEL), (1, 0, 2))
